# baseline (speedup 1.0000x reference)
import functools

import jax
import jax.numpy as jnp
from jax import lax
from jax.experimental import pallas as pl
from jax.experimental.pallas import tpu as pltpu
from jax.experimental.pallas import tpu_sc as plsc

_B = 16384
_K = 8192
_D = 32
_TB = 256
_KC = 2048

_NC = 2
_NS = 16
_NW = _NC * _NS
_BPW = _B // _NW
_IC = 128


def _row_sum32(v):
    b = ((v[:, :8] + v[:, 8:16]) + v[:, 16:24]) + v[:, 24:32]
    c = b[:, :4] + b[:, 4:8]
    e = c[:, :2] + c[:, 2:4]
    return e[:, 0:1] + e[:, 1:2]


def _bf16_round(x):
    return x.astype(jnp.bfloat16).astype(jnp.float32)


def _argmin_loss_body(z_ref, cb_ref, idx_ref, loss_ref):
    z = z_ref[...]
    x_sq = _row_sum32(z * z)
    vals, idxs = [], []
    for c in range(_K // _KC):
        cb = cb_ref[c * _KC:(c + 1) * _KC, :]
        c_sq = _row_sum32(cb * cb)[:, 0][None, :]
        dots = lax.dot_general(z, cb, (((1,), (1,)), ((), ())))
        sq = x_sq + c_sq - 2.0 * dots
        dist = jnp.sqrt(jnp.maximum(sq, 0.0))
        loc_min = jnp.min(dist, axis=-1, keepdims=True)
        kio = lax.broadcasted_iota(jnp.int32, dist.shape, 1)
        loc_idx = jnp.min(jnp.where(dist == loc_min, kio, _K),
                          axis=-1, keepdims=True) + c * _KC
        vals.append(loc_min)
        idxs.append(loc_idx)

    def exact_merge(v0, i0, v1, i1):
        w = v1 < v0
        return jnp.where(w, v1, v0), jnp.where(w, i1, i0)

    nch = _K // _KC
    half = nch // 2
    va, ia = vals[0], idxs[0]
    for c in range(1, half):
        va, ia = exact_merge(va, ia, vals[c], idxs[c])
    vb, ib = vals[half], idxs[half]
    for c in range(half + 1, nch):
        vb, ib = exact_merge(vb, ib, vals[c], idxs[c])
    upd = vb < _bf16_round(va)
    run_idx = jnp.where(upd, ib, ia)
    run_val = jnp.where(upd, vb, va)

    idx_ref[0, 0, :] = run_idx[:, 0]

    @pl.when(pl.program_id(0) == 0)
    def _init():
        loss_ref[...] = jnp.zeros((1, 1), jnp.float32)

    loss_ref[...] += jnp.sum(run_val * run_val).reshape(1, 1)


def _tc_argmin_loss(z_e, codebook):
    return pl.pallas_call(
        _argmin_loss_body,
        grid=(_B // _TB,),
        in_specs=[
            pl.BlockSpec((_TB, _D), lambda i: (i, 0)),
            pl.BlockSpec((_K, _D), lambda i: (0, 0)),
        ],
        out_specs=[
            pl.BlockSpec((1, 1, _TB), lambda i: (i, 0, 0)),
            pl.BlockSpec((1, 1), lambda i: (0, 0)),
        ],
        out_shape=[
            jax.ShapeDtypeStruct((_B // _TB, 1, _TB), jnp.int32),
            jax.ShapeDtypeStruct((1, 1), jnp.float32),
        ],
        compiler_params=pltpu.CompilerParams(
            dimension_semantics=("arbitrary",)),
    )(z_e, codebook)


_DP = 128


def _sc_gather_body(table_hbm, idx_hbm, out_hbm, idx_v, rows_v, sem):
    wid = lax.axis_index("s") * _NC + lax.axis_index("c")
    base = wid * _BPW
    pltpu.sync_copy(idx_hbm.at[pl.ds(wid * (_BPW // _IC), _BPW // _IC)],
                    idx_v)
    copies = [
        pltpu.async_copy(table_hbm.at[idx_v.at[j]],
                         rows_v.at[pl.ds(j * _IC, _IC)], sem)
        for j in range(_BPW // _IC)
    ]
    for c in copies:
        c.wait()
    pltpu.sync_copy(rows_v, out_hbm.at[pl.ds(base, _BPW)])


def _sc_gather(table_padded, idx):
    mesh = plsc.VectorSubcoreMesh(core_axis_name="c", subcore_axis_name="s")
    f = pl.kernel(
        _sc_gather_body,
        out_type=jax.ShapeDtypeStruct((_B, _DP), jnp.float32),
        mesh=mesh,
        scratch_types=[
            pltpu.VMEM((_BPW // _IC, _IC), jnp.int32),
            pltpu.VMEM((_BPW, _DP), jnp.float32),
            pltpu.SemaphoreType.DMA,
        ],
    )
    return f(table_padded, idx.reshape(_B // _IC, _IC))


def kernel(z_e, codebook):
    idx3, sq_sum = _tc_argmin_loss(z_e, codebook)
    idx = idx3.reshape(_B)
    table_padded = jnp.pad(codebook, ((0, 0), (0, _DP - _D)))
    z_q = _sc_gather(table_padded, idx)[:, :_D]
    commitment_loss = (sq_sum[0, 0] / (_B * _D)).astype(jnp.float32)
    return (z_q, commitment_loss)


# speedup vs baseline: 2.2603x; 2.2603x over previous
import functools

import jax
import jax.numpy as jnp
from jax import lax
from jax.experimental import pallas as pl
from jax.experimental.pallas import tpu as pltpu
from jax.experimental.pallas import tpu_sc as plsc

_B = 16384
_K = 8192
_D = 32
_TB = 512
_KC = 2048

_NC = 2
_NS = 16
_NW = _NC * _NS
_BPW = _B // _NW
_IC = 128


def _row_sum32(v):
    b = ((v[:, :8] + v[:, 8:16]) + v[:, 16:24]) + v[:, 24:32]
    c = b[:, :4] + b[:, 4:8]
    e = c[:, :2] + c[:, 2:4]
    return e[:, 0:1] + e[:, 1:2]


def _bf16_round(x):
    return x.astype(jnp.bfloat16).astype(jnp.float32)


def _argmin_loss_body(z_ref, cb_ref, idx_ref, loss_ref, csq_ref):
    @pl.when(pl.program_id(0) == 0)
    def _fill_csq():
        for c in range(_K // _KC):
            cb = cb_ref[c * _KC:(c + 1) * _KC, :]
            csq_ref[0:1, c * _KC:(c + 1) * _KC] = (
                _row_sum32(cb * cb)[:, 0][None, :])

    z = z_ref[...]
    x_sq = _row_sum32(z * z)
    vals, idxs = [], []
    for c in range(_K // _KC):
        cb = cb_ref[c * _KC:(c + 1) * _KC, :]
        c_sq = csq_ref[0:1, c * _KC:(c + 1) * _KC]
        dots = lax.dot_general(z, cb, (((1,), (1,)), ((), ())))
        sq = x_sq + c_sq - 2.0 * dots
        dist = jnp.sqrt(jnp.maximum(sq, 0.0))
        loc_min = jnp.min(dist, axis=-1, keepdims=True)
        kio = lax.broadcasted_iota(jnp.int32, dist.shape, 1)
        loc_idx = jnp.min(jnp.where(dist == loc_min, kio, _K),
                          axis=-1, keepdims=True) + c * _KC
        vals.append(loc_min)
        idxs.append(loc_idx)

    def exact_merge(v0, i0, v1, i1):
        w = v1 < v0
        return jnp.where(w, v1, v0), jnp.where(w, i1, i0)

    nch = _K // _KC
    half = nch // 2
    va, ia = vals[0], idxs[0]
    for c in range(1, half):
        va, ia = exact_merge(va, ia, vals[c], idxs[c])
    vb, ib = vals[half], idxs[half]
    for c in range(half + 1, nch):
        vb, ib = exact_merge(vb, ib, vals[c], idxs[c])
    upd = vb < _bf16_round(va)
    run_idx = jnp.where(upd, ib, ia)
    run_val = jnp.where(upd, vb, va)

    idx_ref[0, 0, :] = run_idx[:, 0]

    @pl.when(pl.program_id(0) == 0)
    def _init():
        loss_ref[...] = jnp.zeros((1, 1), jnp.float32)

    loss_ref[...] += jnp.sum(run_val * run_val).reshape(1, 1)


def _tc_argmin_loss(z_e, codebook):
    return pl.pallas_call(
        _argmin_loss_body,
        grid=(_B // _TB,),
        in_specs=[
            pl.BlockSpec((_TB, _D), lambda i: (i, 0)),
            pl.BlockSpec((_K, _D), lambda i: (0, 0)),
        ],
        out_specs=[
            pl.BlockSpec((1, 1, _TB), lambda i: (i, 0, 0)),
            pl.BlockSpec((1, 1), lambda i: (0, 0)),
        ],
        out_shape=[
            jax.ShapeDtypeStruct((_B // _TB, 1, _TB), jnp.int32),
            jax.ShapeDtypeStruct((1, 1), jnp.float32),
        ],
        scratch_shapes=[pltpu.VMEM((1, _K), jnp.float32)],
        compiler_params=pltpu.CompilerParams(
            dimension_semantics=("arbitrary",)),
    )(z_e, codebook)


_DP = 128


def _sc_gather_body(table_hbm, idx_hbm, out_hbm, idx_v, rows_v, sem):
    wid = lax.axis_index("s") * _NC + lax.axis_index("c")
    base = wid * _BPW
    pltpu.sync_copy(idx_hbm.at[pl.ds(wid * (_BPW // _IC), _BPW // _IC)],
                    idx_v)
    copies = [
        pltpu.async_copy(table_hbm.at[idx_v.at[j]],
                         rows_v.at[pl.ds(j * _IC, _IC)], sem)
        for j in range(_BPW // _IC)
    ]
    for c in copies:
        c.wait()
    pltpu.sync_copy(rows_v, out_hbm.at[pl.ds(base, _BPW)])


def _sc_gather(table_padded, idx):
    mesh = plsc.VectorSubcoreMesh(core_axis_name="c", subcore_axis_name="s")
    f = pl.kernel(
        _sc_gather_body,
        out_type=jax.ShapeDtypeStruct((_B, _DP), jnp.float32),
        mesh=mesh,
        scratch_types=[
            pltpu.VMEM((_BPW // _IC, _IC), jnp.int32),
            pltpu.VMEM((_BPW, _DP), jnp.float32),
            pltpu.SemaphoreType.DMA,
        ],
    )
    return f(table_padded, idx.reshape(_B // _IC, _IC))


def kernel(z_e, codebook):
    idx3, sq_sum = _tc_argmin_loss(z_e, codebook)
    idx = idx3.reshape(_B)
    table_padded = jnp.pad(codebook, ((0, 0), (0, _DP - _D)))
    z_q = _sc_gather(table_padded, idx)[:, :_D]
    commitment_loss = (sq_sum[0, 0] / (_B * _D)).astype(jnp.float32)
    return (z_q, commitment_loss)


# speedup vs baseline: 2.4319x; 1.0759x over previous
import functools

import jax
import jax.numpy as jnp
from jax import lax
from jax.experimental import pallas as pl
from jax.experimental.pallas import tpu as pltpu
from jax.experimental.pallas import tpu_sc as plsc

_B = 16384
_K = 8192
_D = 32
_TB = 512
_KC = 2048

_NC = 2
_NS = 16
_NW = _NC * _NS
_BPW = _B // _NW
_IC = 128


def _row_sum32(v):
    b = ((v[:, :8] + v[:, 8:16]) + v[:, 16:24]) + v[:, 24:32]
    c = b[:, :4] + b[:, 4:8]
    e = c[:, :2] + c[:, 2:4]
    return e[:, 0:1] + e[:, 1:2]


def _bf16_round(x):
    return x.astype(jnp.bfloat16).astype(jnp.float32)


def _argmin_loss_body(z_ref, cb_ref, idx_ref, loss_ref, csq_ref):
    @pl.when(pl.program_id(0) == 0)
    def _fill_csq():
        for c in range(_K // _KC):
            cb = cb_ref[c * _KC:(c + 1) * _KC, :]
            csq_ref[0:1, c * _KC:(c + 1) * _KC] = (
                _row_sum32(cb * cb)[:, 0][None, :])

    z = z_ref[...]
    x_sq = _row_sum32(z * z)
    vals, idxs = [], []
    for c in range(_K // _KC):
        cb = cb_ref[c * _KC:(c + 1) * _KC, :]
        c_sq = csq_ref[0:1, c * _KC:(c + 1) * _KC]
        dots = lax.dot_general(z, cb, (((1,), (1,)), ((), ())))
        sq = x_sq + c_sq - 2.0 * dots
        dist = jnp.sqrt(jnp.maximum(sq, 0.0))
        acc_v = dist[:, 0:128]
        acc_j = jnp.zeros((_TB, 128), jnp.int32)
        for jj in range(1, _KC // 128):
            v = dist[:, jj * 128:(jj + 1) * 128]
            upd = v < acc_v
            acc_j = jnp.where(upd, jj, acc_j)
            acc_v = jnp.where(upd, v, acc_v)
        loc_min = jnp.min(acc_v, axis=-1, keepdims=True)
        lane = lax.broadcasted_iota(jnp.int32, (_TB, 128), 1)
        j_full = acc_j * 128 + lane
        loc_idx = jnp.min(jnp.where(acc_v == loc_min, j_full, _K),
                          axis=-1, keepdims=True) + c * _KC
        vals.append(loc_min)
        idxs.append(loc_idx)

    def exact_merge(v0, i0, v1, i1):
        w = v1 < v0
        return jnp.where(w, v1, v0), jnp.where(w, i1, i0)

    nch = _K // _KC
    half = nch // 2
    va, ia = vals[0], idxs[0]
    for c in range(1, half):
        va, ia = exact_merge(va, ia, vals[c], idxs[c])
    vb, ib = vals[half], idxs[half]
    for c in range(half + 1, nch):
        vb, ib = exact_merge(vb, ib, vals[c], idxs[c])
    upd = vb < _bf16_round(va)
    run_idx = jnp.where(upd, ib, ia)
    run_val = jnp.where(upd, vb, va)

    idx_ref[0, 0, :] = run_idx[:, 0]

    @pl.when(pl.program_id(0) == 0)
    def _init():
        loss_ref[...] = jnp.zeros((1, 1), jnp.float32)

    loss_ref[...] += jnp.sum(run_val * run_val).reshape(1, 1)


def _tc_argmin_loss(z_e, codebook):
    return pl.pallas_call(
        _argmin_loss_body,
        grid=(_B // _TB,),
        in_specs=[
            pl.BlockSpec((_TB, _D), lambda i: (i, 0)),
            pl.BlockSpec((_K, _D), lambda i: (0, 0)),
        ],
        out_specs=[
            pl.BlockSpec((1, 1, _TB), lambda i: (i, 0, 0)),
            pl.BlockSpec((1, 1), lambda i: (0, 0)),
        ],
        out_shape=[
            jax.ShapeDtypeStruct((_B // _TB, 1, _TB), jnp.int32),
            jax.ShapeDtypeStruct((1, 1), jnp.float32),
        ],
        scratch_shapes=[pltpu.VMEM((1, _K), jnp.float32)],
        compiler_params=pltpu.CompilerParams(
            dimension_semantics=("arbitrary",)),
    )(z_e, codebook)


_DP = 128


def _sc_gather_body(table_hbm, idx_hbm, out_hbm, idx_v, rows_v, sem):
    wid = lax.axis_index("s") * _NC + lax.axis_index("c")
    base = wid * _BPW
    pltpu.sync_copy(idx_hbm.at[pl.ds(wid * (_BPW // _IC), _BPW // _IC)],
                    idx_v)
    copies = [
        pltpu.async_copy(table_hbm.at[idx_v.at[j]],
                         rows_v.at[pl.ds(j * _IC, _IC)], sem)
        for j in range(_BPW // _IC)
    ]
    for c in copies:
        c.wait()
    pltpu.sync_copy(rows_v, out_hbm.at[pl.ds(base, _BPW)])


def _sc_gather(table_padded, idx):
    mesh = plsc.VectorSubcoreMesh(core_axis_name="c", subcore_axis_name="s")
    f = pl.kernel(
        _sc_gather_body,
        out_type=jax.ShapeDtypeStruct((_B, _DP), jnp.float32),
        mesh=mesh,
        scratch_types=[
            pltpu.VMEM((_BPW // _IC, _IC), jnp.int32),
            pltpu.VMEM((_BPW, _DP), jnp.float32),
            pltpu.SemaphoreType.DMA,
        ],
    )
    return f(table_padded, idx.reshape(_B // _IC, _IC))


def kernel(z_e, codebook):
    idx3, sq_sum = _tc_argmin_loss(z_e, codebook)
    idx = idx3.reshape(_B)
    table_padded = jnp.pad(codebook, ((0, 0), (0, _DP - _D)))
    z_q = _sc_gather(table_padded, idx)[:, :_D]
    commitment_loss = (sq_sum[0, 0] / (_B * _D)).astype(jnp.float32)
    return (z_q, commitment_loss)


# speedup vs baseline: 2.4719x; 1.0164x over previous
import functools

import jax
import jax.numpy as jnp
from jax import lax
from jax.experimental import pallas as pl
from jax.experimental.pallas import tpu as pltpu
from jax.experimental.pallas import tpu_sc as plsc

_B = 16384
_K = 8192
_D = 32
_TB = 1024
_KC = 2048

_NC = 2
_NS = 16
_NW = _NC * _NS
_BPW = _B // _NW
_IC = 128


def _row_sum32(v):
    b = ((v[:, :8] + v[:, 8:16]) + v[:, 16:24]) + v[:, 24:32]
    c = b[:, :4] + b[:, 4:8]
    e = c[:, :2] + c[:, 2:4]
    return e[:, 0:1] + e[:, 1:2]


def _bf16_round(x):
    return x.astype(jnp.bfloat16).astype(jnp.float32)


def _argmin_loss_body(z_ref, cb_ref, idx_ref, loss_ref, csq_ref):
    @pl.when(pl.program_id(0) == 0)
    def _fill_csq():
        for c in range(_K // _KC):
            cb = cb_ref[c * _KC:(c + 1) * _KC, :]
            csq_ref[0:1, c * _KC:(c + 1) * _KC] = (
                _row_sum32(cb * cb)[:, 0][None, :])

    z = z_ref[...]
    x_sq = _row_sum32(z * z)
    vals, idxs = [], []
    for c in range(_K // _KC):
        cb = cb_ref[c * _KC:(c + 1) * _KC, :]
        c_sq = csq_ref[0:1, c * _KC:(c + 1) * _KC]
        dots = lax.dot_general(z, cb, (((1,), (1,)), ((), ())))
        sq = x_sq + c_sq - 2.0 * dots
        dist = jnp.sqrt(jnp.maximum(sq, 0.0))
        acc_v = dist[:, 0:128]
        acc_j = jnp.zeros((_TB, 128), jnp.int32)
        for jj in range(1, _KC // 128):
            v = dist[:, jj * 128:(jj + 1) * 128]
            upd = v < acc_v
            acc_j = jnp.where(upd, jj, acc_j)
            acc_v = jnp.minimum(acc_v, v)
        loc_min = jnp.min(acc_v, axis=-1, keepdims=True)
        lane = lax.broadcasted_iota(jnp.int32, (_TB, 128), 1)
        j_full = acc_j * 128 + lane
        loc_idx = jnp.min(jnp.where(acc_v == loc_min, j_full, _K),
                          axis=-1, keepdims=True) + c * _KC
        vals.append(loc_min)
        idxs.append(loc_idx)

    def exact_merge(v0, i0, v1, i1):
        w = v1 < v0
        return jnp.where(w, v1, v0), jnp.where(w, i1, i0)

    nch = _K // _KC
    half = nch // 2
    va, ia = vals[0], idxs[0]
    for c in range(1, half):
        va, ia = exact_merge(va, ia, vals[c], idxs[c])
    vb, ib = vals[half], idxs[half]
    for c in range(half + 1, nch):
        vb, ib = exact_merge(vb, ib, vals[c], idxs[c])
    upd = vb < _bf16_round(va)
    run_idx = jnp.where(upd, ib, ia)
    run_val = jnp.where(upd, vb, va)

    idx_ref[0, 0, :] = run_idx[:, 0]

    @pl.when(pl.program_id(0) == 0)
    def _init():
        loss_ref[...] = jnp.zeros((1, 1), jnp.float32)

    loss_ref[...] += jnp.sum(run_val * run_val).reshape(1, 1)


def _tc_argmin_loss(z_e, codebook):
    return pl.pallas_call(
        _argmin_loss_body,
        grid=(_B // _TB,),
        in_specs=[
            pl.BlockSpec((_TB, _D), lambda i: (i, 0)),
            pl.BlockSpec((_K, _D), lambda i: (0, 0)),
        ],
        out_specs=[
            pl.BlockSpec((1, 1, _TB), lambda i: (i, 0, 0)),
            pl.BlockSpec((1, 1), lambda i: (0, 0)),
        ],
        out_shape=[
            jax.ShapeDtypeStruct((_B // _TB, 1, _TB), jnp.int32),
            jax.ShapeDtypeStruct((1, 1), jnp.float32),
        ],
        scratch_shapes=[pltpu.VMEM((1, _K), jnp.float32)],
        compiler_params=pltpu.CompilerParams(
            dimension_semantics=("arbitrary",)),
    )(z_e, codebook)


_DP = 128


def _sc_gather_body(table_hbm, idx_hbm, out_hbm, idx_v, rows_v, sem):
    wid = lax.axis_index("s") * _NC + lax.axis_index("c")
    base = wid * _BPW
    pltpu.sync_copy(idx_hbm.at[pl.ds(wid * (_BPW // _IC), _BPW // _IC)],
                    idx_v)
    copies = [
        pltpu.async_copy(table_hbm.at[idx_v.at[j]],
                         rows_v.at[pl.ds(j * _IC, _IC)], sem)
        for j in range(_BPW // _IC)
    ]
    for c in copies:
        c.wait()
    pltpu.sync_copy(rows_v, out_hbm.at[pl.ds(base, _BPW)])


def _sc_gather(table_padded, idx):
    mesh = plsc.VectorSubcoreMesh(core_axis_name="c", subcore_axis_name="s")
    f = pl.kernel(
        _sc_gather_body,
        out_type=jax.ShapeDtypeStruct((_B, _DP), jnp.float32),
        mesh=mesh,
        scratch_types=[
            pltpu.VMEM((_BPW // _IC, _IC), jnp.int32),
            pltpu.VMEM((_BPW, _DP), jnp.float32),
            pltpu.SemaphoreType.DMA,
        ],
    )
    return f(table_padded, idx.reshape(_B // _IC, _IC))


def kernel(z_e, codebook):
    idx3, sq_sum = _tc_argmin_loss(z_e, codebook)
    idx = idx3.reshape(_B)
    table_padded = jnp.pad(codebook, ((0, 0), (0, _DP - _D)))
    z_q = _sc_gather(table_padded, idx)[:, :_D]
    commitment_loss = (sq_sum[0, 0] / (_B * _D)).astype(jnp.float32)
    return (z_q, commitment_loss)
